# baseline (speedup 1.0000x reference)
import math

import jax
import jax.numpy as jnp
import numpy as np
from jax.experimental import pallas as pl
from jax.experimental.pallas import tpu as pltpu
from jax.experimental.pallas import tpu_sc as plsc

_PERM_HD = np.concatenate([np.arange(0, 128, 2), np.arange(1, 128, 2)])
_PERM_C = np.concatenate([h * 128 + _PERM_HD for h in range(6)])

_B, _S, _C, _HD = 1, 4096, 768, 128
_H = _C // _HD
_S1 = _S // 64
_PTHR = 0.9
_NEG = -1e30
_BQ = 1024
_BX = 512


def _qkv_body(x_ref, wq_ref, bq_ref, wk_ref, bk_ref, wv_ref, bv_ref,
              gq_ref, gk_ref, r00_ref, r01_ref, r10_ref, r11_ref,
              q_ref, k_ref, v_ref):
    x = x_ref[...]
    eps = jnp.finfo(jnp.float32).eps

    def norm_rope(t, g_ref):
        t3 = t.reshape(_BX, _H, _HD)
        ms = jnp.mean(t3 * t3, axis=-1, keepdims=True)
        t3 = t3 * jax.lax.rsqrt(ms + eps) * g_ref[...].reshape(1, 1, _HD)
        e = t3[:, :, : _HD // 2]
        o = t3[:, :, _HD // 2:]
        r00 = r00_ref[...][:, None, :]
        r01 = r01_ref[...][:, None, :]
        r10 = r10_ref[...][:, None, :]
        r11 = r11_ref[...][:, None, :]
        oe = r00 * e + r01 * o
        oo = r10 * e + r11 * o
        return jnp.concatenate([oe, oo], axis=-1).reshape(_BX, _C)

    xb = x.astype(jnp.bfloat16)
    q = jnp.dot(xb, wq_ref[...], preferred_element_type=jnp.float32) + bq_ref[...]
    k = jnp.dot(xb, wk_ref[...], preferred_element_type=jnp.float32) + bk_ref[...]
    v = jnp.dot(xb, wv_ref[...], preferred_element_type=jnp.float32) + bv_ref[...]
    q_ref[...] = norm_rope(q, gq_ref)
    k_ref[...] = norm_rope(k, gk_ref)
    v_ref[...] = v


def _amap_body(q_ref, k_ref, a_ref):
    q = q_ref[...]
    k = k_ref[...]
    qa = jnp.mean(q.reshape(_S1, 64, _HD), axis=1)
    ka = jnp.mean(k.reshape(_S1, 64, _HD), axis=1)
    logits = jax.lax.dot_general(
        qa, ka, (((1,), (1,)), ((), ())),
        preferred_element_type=jnp.float32) * (1.0 / math.sqrt(_HD))
    a_ref[0] = jax.nn.softmax(logits, axis=-1)


_RPW = 16
_NW = 32
_SCROWS = _RPW * _NW


def _sc_vsort(a):
    k, i = plsc.sort_key_val(a[0], a[1])
    return (k, i)


def _sc_rev(a):
    return (jax.lax.rev(a[0], (0,)), jax.lax.rev(a[1], (0,)))


def _sc_cmpx(a, b):
    m = a[0] <= b[0]
    lo = (jnp.where(m, a[0], b[0]), jnp.where(m, a[1], b[1]))
    hi = (jnp.where(m, b[0], a[0]), jnp.where(m, b[1], a[1]))
    return lo, hi


def _sc_sort64(vs):
    s = [_sc_vsort(v) for v in vs]
    a_lo, a_hi = _sc_cmpx(s[0], _sc_rev(s[1]))
    b_lo, b_hi = _sc_cmpx(s[2], _sc_rev(s[3]))
    a = (_sc_vsort(a_lo), _sc_vsort(a_hi))
    b = (_sc_vsort(b_lo), _sc_vsort(b_hi))
    l0, h0 = _sc_cmpx(a[0], _sc_rev(b[1]))
    l1, h1 = _sc_cmpx(a[1], _sc_rev(b[0]))
    out = []
    for x0, x1 in ((l0, l1), (h0, h1)):
        p, q = _sc_cmpx(x0, x1)
        out.append(_sc_vsort(p))
        out.append(_sc_vsort(q))
    return out


def _sc_mask_body(amap_hbm, out_hbm, amap_v, bias_v):
    wid = jax.lax.axis_index("s") * 2 + jax.lax.axis_index("c")
    base = wid * _RPW
    pltpu.sync_copy(amap_hbm.at[pl.ds(base, _RPW)], amap_v)
    thr = 1.0 - _PTHR
    for r in range(_RPW):
        vs = [(amap_v[r, pl.ds(16 * t, 16)],
               jax.lax.iota(jnp.int32, 16) + 16 * t) for t in range(4)]
        srt = _sc_sort64(vs)
        carry = 0.0
        for t in range(4):
            k, i = srt[t]
            c = plsc.cumsum(k) + carry
            carry = carry + jnp.sum(k)
            vals = jnp.where(c >= thr, 0.0, _NEG)
            plsc.store_scatter(
                bias_v, [jnp.full((16,), r, jnp.int32), i], vals)
    pltpu.sync_copy(bias_v, out_hbm.at[pl.ds(base, _RPW)])


def _expand_body(cb_ref, sta_ref, bias_ref):
    cb = jnp.where(sta_ref[0] > 0.0, 0.0, cb_ref[0])
    bcol = cb.astype(jnp.bfloat16).T
    bias_ref[0] = jnp.broadcast_to(
        bcol[:, None, :], (_S1, 64, _S1)).reshape(_S, _S1)


def _attn_body(q_ref, k_ref, v_ref, bias_ref, o_ref):
    i = pl.program_id(1)
    nb = _BQ // 64
    q = q_ref[...].astype(jnp.bfloat16)
    k = k_ref[...].astype(jnp.bfloat16)
    v = v_ref[...].astype(jnp.bfloat16)
    r_blk = jax.lax.broadcasted_iota(jnp.int32, (_BQ, _S1), 0) // 64 + i * nb
    c_idx = jax.lax.broadcasted_iota(jnp.int32, (_BQ, _S1), 1)
    onehot = (c_idx == r_blk).astype(jnp.bfloat16)
    q_aug = jnp.concatenate([q, onehot], axis=1)
    k_aug = jnp.concatenate([k, bias_ref[0]], axis=1)
    s = jax.lax.dot_general(
        q_aug, k_aug, (((1,), (1,)), ((), ())),
        preferred_element_type=jnp.float32)
    m = jnp.max(s, axis=-1, keepdims=True)
    p = jnp.exp((s - m) * (1.0 / math.sqrt(_HD)))
    l = jnp.sum(p, axis=-1, keepdims=True)
    o = jnp.dot(p.astype(jnp.bfloat16), v,
                preferred_element_type=jnp.float32) / l
    o_ref[...] = o


def _proj_body(x_ref, w_ref, b_ref, o_ref):
    o_ref[...] = (jnp.dot(x_ref[...].astype(jnp.bfloat16), w_ref[...],
                          preferred_element_type=jnp.float32) + b_ref[...])


def kernel(x, rope, sta_mask, Wq, bq, Wk, bk, Wv, bv, gq, gk, Wo, bo):
    f32 = jnp.float32
    x2 = x.reshape(_S, _C)
    rr = rope.reshape(_S, _HD // 2, 2, 2)
    r00, r01 = rr[:, :, 0, 0], rr[:, :, 0, 1]
    r10, r11 = rr[:, :, 1, 0], rr[:, :, 1, 1]
    staf = sta_mask.reshape(_H, _S1, _S1).astype(f32)

    full_cc = pl.BlockSpec((_C, _C), lambda i: (0, 0))
    row_c = pl.BlockSpec((1, _C), lambda i: (0, 0))
    tile_x = pl.BlockSpec((_BX, _C), lambda i: (i, 0))
    tile_r = pl.BlockSpec((_BX, _HD // 2), lambda i: (i, 0))
    g_spec = pl.BlockSpec((1, _HD), lambda i: (0, 0))

    q2, k2, v2 = pl.pallas_call(
        _qkv_body,
        grid=(_S // _BX,),
        in_specs=[tile_x, full_cc, row_c, full_cc, row_c, full_cc, row_c,
                  g_spec, g_spec, tile_r, tile_r, tile_r, tile_r],
        out_specs=[tile_x, tile_x, tile_x],
        out_shape=[jax.ShapeDtypeStruct((_S, _C), f32)] * 3,
    )(x2, Wq.T[:, _PERM_C].astype(jnp.bfloat16), bq[_PERM_C].reshape(1, _C),
      Wk.T[:, _PERM_C].astype(jnp.bfloat16), bk[_PERM_C].reshape(1, _C),
      Wv.T.astype(jnp.bfloat16), bv.reshape(1, _C),
      gq[_PERM_HD].reshape(1, _HD), gk[_PERM_HD].reshape(1, _HD),
      r00, r01, r10, r11)

    head_col = pl.BlockSpec((_S, _HD), lambda h: (0, h))
    blk_spec = pl.BlockSpec((1, _S1, _S1), lambda h: (h, 0, 0))
    bias_spec = pl.BlockSpec((1, _S, _S1), lambda h: (h, 0, 0))
    amap = pl.pallas_call(
        _amap_body,
        grid=(_H,),
        in_specs=[head_col, head_col],
        out_specs=blk_spec,
        out_shape=jax.ShapeDtypeStruct((_H, _S1, _S1), f32),
    )(q2, k2)

    amap_pad = jnp.zeros((_SCROWS, _S1), f32)
    amap_pad = amap_pad.at[: _H * _S1].set(amap.reshape(_H * _S1, _S1))
    cb = pl.kernel(
        _sc_mask_body,
        out_type=jax.ShapeDtypeStruct((_SCROWS, _S1), f32),
        mesh=plsc.VectorSubcoreMesh(core_axis_name="c", subcore_axis_name="s"),
        scratch_types=[pltpu.VMEM((_RPW, _S1), f32),
                       pltpu.VMEM((_RPW, _S1), f32)],
        compiler_params=pltpu.CompilerParams(needs_layout_passes=False),
    )(amap_pad)

    bias = pl.pallas_call(
        _expand_body,
        grid=(_H,),
        in_specs=[blk_spec, blk_spec],
        out_specs=bias_spec,
        out_shape=jax.ShapeDtypeStruct((_H, _S, _S1), jnp.bfloat16),
    )(cb[: _H * _S1].reshape(_H, _S1, _S1), staf)

    att = pl.pallas_call(
        _attn_body,
        grid=(_H, _S // _BQ),
        in_specs=[pl.BlockSpec((_BQ, _HD), lambda h, i: (i, h)),
                  pl.BlockSpec((_S, _HD), lambda h, i: (0, h)),
                  pl.BlockSpec((_S, _HD), lambda h, i: (0, h)),
                  pl.BlockSpec((1, _S, _S1), lambda h, i: (h, 0, 0))],
        out_specs=pl.BlockSpec((_BQ, _HD), lambda h, i: (i, h)),
        out_shape=jax.ShapeDtypeStruct((_S, _C), f32),
    )(q2, k2, v2, bias)

    out = pl.pallas_call(
        _proj_body,
        grid=(_S // _BX,),
        in_specs=[tile_x, full_cc, row_c],
        out_specs=tile_x,
        out_shape=jax.ShapeDtypeStruct((_S, _C), f32),
    )(att, Wo.T.astype(jnp.bfloat16), bo.reshape(1, _C))

    return out.reshape(_B, _S, _C)


# speedup vs baseline: 1.0246x; 1.0246x over previous
import math

import jax
import jax.numpy as jnp
import numpy as np
from jax.experimental import pallas as pl
from jax.experimental.pallas import tpu as pltpu
from jax.experimental.pallas import tpu_sc as plsc

_PERM_HD = np.concatenate([np.arange(0, 128, 2), np.arange(1, 128, 2)])
_PERM_C = np.concatenate([h * 128 + _PERM_HD for h in range(6)])

_B, _S, _C, _HD = 1, 4096, 768, 128
_H = _C // _HD
_S1 = _S // 64
_PTHR = 0.9
_NEG = -1e30
_BQ = 1024
_BX = 512


def _qkv_body(x_ref, wq_ref, bq_ref, wk_ref, bk_ref, wv_ref, bv_ref,
              gq_ref, gk_ref, r00_ref, r01_ref, r10_ref, r11_ref,
              q_ref, k_ref, v_ref, qa_ref, ka_ref):
    x = x_ref[...]
    eps = jnp.finfo(jnp.float32).eps

    def norm_rope(t, g_ref):
        t3 = t.reshape(_BX, _H, _HD)
        ms = jnp.mean(t3 * t3, axis=-1, keepdims=True)
        t3 = t3 * jax.lax.rsqrt(ms + eps) * g_ref[...].reshape(1, 1, _HD)
        e = t3[:, :, : _HD // 2]
        o = t3[:, :, _HD // 2:]
        r00 = r00_ref[...][:, None, :]
        r01 = r01_ref[...][:, None, :]
        r10 = r10_ref[...][:, None, :]
        r11 = r11_ref[...][:, None, :]
        oe = r00 * e + r01 * o
        oo = r10 * e + r11 * o
        return jnp.concatenate([oe, oo], axis=-1).reshape(_BX, _C)

    xb = x.astype(jnp.bfloat16)
    q = jnp.dot(xb, wq_ref[...], preferred_element_type=jnp.float32) + bq_ref[...]
    k = jnp.dot(xb, wk_ref[...], preferred_element_type=jnp.float32) + bk_ref[...]
    v = jnp.dot(xb, wv_ref[...], preferred_element_type=jnp.float32) + bv_ref[...]
    qf = norm_rope(q, gq_ref)
    kf = norm_rope(k, gk_ref)
    q_ref[...] = qf.astype(jnp.bfloat16)
    k_ref[...] = kf.astype(jnp.bfloat16)
    v_ref[...] = v.astype(jnp.bfloat16)
    qa_ref[...] = jnp.mean(qf.reshape(_BX // 64, 64, _C), axis=1)
    ka_ref[...] = jnp.mean(kf.reshape(_BX // 64, 64, _C), axis=1)


def _amap_body(qa_ref, ka_ref, a_ref):
    qa = qa_ref[...]
    ka = ka_ref[...]
    logits = jax.lax.dot_general(
        qa, ka, (((1,), (1,)), ((), ())),
        preferred_element_type=jnp.float32) * (1.0 / math.sqrt(_HD))
    a_ref[0] = jax.nn.softmax(logits, axis=-1)


_RPW = 16
_NW = 32
_SCROWS = _RPW * _NW


def _sc_vsort(a):
    k, i = plsc.sort_key_val(a[0], a[1])
    return (k, i)


def _sc_rev(a):
    return (jax.lax.rev(a[0], (0,)), jax.lax.rev(a[1], (0,)))


def _sc_cmpx(a, b):
    m = a[0] <= b[0]
    lo = (jnp.where(m, a[0], b[0]), jnp.where(m, a[1], b[1]))
    hi = (jnp.where(m, b[0], a[0]), jnp.where(m, b[1], a[1]))
    return lo, hi


def _sc_sort64(vs):
    s = [_sc_vsort(v) for v in vs]
    a_lo, a_hi = _sc_cmpx(s[0], _sc_rev(s[1]))
    b_lo, b_hi = _sc_cmpx(s[2], _sc_rev(s[3]))
    a = (_sc_vsort(a_lo), _sc_vsort(a_hi))
    b = (_sc_vsort(b_lo), _sc_vsort(b_hi))
    l0, h0 = _sc_cmpx(a[0], _sc_rev(b[1]))
    l1, h1 = _sc_cmpx(a[1], _sc_rev(b[0]))
    out = []
    for x0, x1 in ((l0, l1), (h0, h1)):
        p, q = _sc_cmpx(x0, x1)
        out.append(_sc_vsort(p))
        out.append(_sc_vsort(q))
    return out


def _sc_mask_body(amap_hbm, out_hbm, amap_v, bias_v):
    wid = jax.lax.axis_index("s") * 2 + jax.lax.axis_index("c")
    base = wid * _RPW
    pltpu.sync_copy(amap_hbm.at[pl.ds(base, _RPW)], amap_v)
    thr = 1.0 - _PTHR
    for r in range(_RPW):
        vs = [(amap_v[r, pl.ds(16 * t, 16)],
               jax.lax.iota(jnp.int32, 16) + 16 * t) for t in range(4)]
        srt = _sc_sort64(vs)
        carry = 0.0
        for t in range(4):
            k, i = srt[t]
            c = plsc.cumsum(k) + carry
            carry = carry + jnp.sum(k)
            vals = jnp.where(c >= thr, 0.0, _NEG)
            plsc.store_scatter(
                bias_v, [jnp.full((16,), r, jnp.int32), i], vals)
    pltpu.sync_copy(bias_v, out_hbm.at[pl.ds(base, _RPW)])


def _expand_body(cb_ref, sta_ref, bias_ref):
    cb = jnp.where(sta_ref[0] > 0.0, 0.0, cb_ref[0])
    bcol = cb.astype(jnp.bfloat16).T
    bias_ref[0] = jnp.broadcast_to(
        bcol[:, None, :], (_S1, 64, _S1)).reshape(_S, _S1)


def _attn_body(q_ref, k_ref, v_ref, bias_ref, o_ref):
    i = pl.program_id(1)
    nb = _BQ // 64
    q = q_ref[...]
    k = k_ref[...]
    v = v_ref[...]
    r_blk = jax.lax.broadcasted_iota(jnp.int32, (_BQ, _S1), 0) // 64 + i * nb
    c_idx = jax.lax.broadcasted_iota(jnp.int32, (_BQ, _S1), 1)
    onehot = (c_idx == r_blk).astype(jnp.bfloat16)
    q_aug = jnp.concatenate([q, onehot], axis=1)
    k_aug = jnp.concatenate([k, bias_ref[0]], axis=1)
    s = jax.lax.dot_general(
        q_aug, k_aug, (((1,), (1,)), ((), ())),
        preferred_element_type=jnp.float32)
    m = jnp.max(s, axis=-1, keepdims=True)
    p = jnp.exp((s - m) * (1.0 / math.sqrt(_HD)))
    l = jnp.sum(p, axis=-1, keepdims=True)
    o = jnp.dot(p.astype(jnp.bfloat16), v,
                preferred_element_type=jnp.float32) / l
    o_ref[...] = o.astype(jnp.bfloat16)


def _proj_body(x_ref, w_ref, b_ref, o_ref):
    o_ref[...] = (jnp.dot(x_ref[...], w_ref[...],
                          preferred_element_type=jnp.float32) + b_ref[...])


def kernel(x, rope, sta_mask, Wq, bq, Wk, bk, Wv, bv, gq, gk, Wo, bo):
    f32 = jnp.float32
    x2 = x.reshape(_S, _C)
    rr = rope.reshape(_S, _HD // 2, 2, 2)
    r00, r01 = rr[:, :, 0, 0], rr[:, :, 0, 1]
    r10, r11 = rr[:, :, 1, 0], rr[:, :, 1, 1]
    staf = sta_mask.reshape(_H, _S1, _S1).astype(f32)

    full_cc = pl.BlockSpec((_C, _C), lambda i: (0, 0))
    row_c = pl.BlockSpec((1, _C), lambda i: (0, 0))
    tile_x = pl.BlockSpec((_BX, _C), lambda i: (i, 0))
    tile_r = pl.BlockSpec((_BX, _HD // 2), lambda i: (i, 0))
    g_spec = pl.BlockSpec((1, _HD), lambda i: (0, 0))

    pool_tile = pl.BlockSpec((_BX // 64, _C), lambda i: (i, 0))
    q2, k2, v2, qa2, ka2 = pl.pallas_call(
        _qkv_body,
        grid=(_S // _BX,),
        in_specs=[tile_x, full_cc, row_c, full_cc, row_c, full_cc, row_c,
                  g_spec, g_spec, tile_r, tile_r, tile_r, tile_r],
        out_specs=[tile_x, tile_x, tile_x, pool_tile, pool_tile],
        out_shape=[jax.ShapeDtypeStruct((_S, _C), jnp.bfloat16)] * 3
        + [jax.ShapeDtypeStruct((_S1, _C), f32)] * 2,
    )(x2, Wq.T[:, _PERM_C].astype(jnp.bfloat16), bq[_PERM_C].reshape(1, _C),
      Wk.T[:, _PERM_C].astype(jnp.bfloat16), bk[_PERM_C].reshape(1, _C),
      Wv.T.astype(jnp.bfloat16), bv.reshape(1, _C),
      gq[_PERM_HD].reshape(1, _HD), gk[_PERM_HD].reshape(1, _HD),
      r00, r01, r10, r11)

    head_col = pl.BlockSpec((_S, _HD), lambda h: (0, h))
    pool_col = pl.BlockSpec((_S1, _HD), lambda h: (0, h))
    blk_spec = pl.BlockSpec((1, _S1, _S1), lambda h: (h, 0, 0))
    bias_spec = pl.BlockSpec((1, _S, _S1), lambda h: (h, 0, 0))
    amap = pl.pallas_call(
        _amap_body,
        grid=(_H,),
        in_specs=[pool_col, pool_col],
        out_specs=blk_spec,
        out_shape=jax.ShapeDtypeStruct((_H, _S1, _S1), f32),
    )(qa2, ka2)

    amap_pad = jnp.zeros((_SCROWS, _S1), f32)
    amap_pad = amap_pad.at[: _H * _S1].set(amap.reshape(_H * _S1, _S1))
    cb = pl.kernel(
        _sc_mask_body,
        out_type=jax.ShapeDtypeStruct((_SCROWS, _S1), f32),
        mesh=plsc.VectorSubcoreMesh(core_axis_name="c", subcore_axis_name="s"),
        scratch_types=[pltpu.VMEM((_RPW, _S1), f32),
                       pltpu.VMEM((_RPW, _S1), f32)],
        compiler_params=pltpu.CompilerParams(needs_layout_passes=False),
    )(amap_pad)

    bias = pl.pallas_call(
        _expand_body,
        grid=(_H,),
        in_specs=[blk_spec, blk_spec],
        out_specs=bias_spec,
        out_shape=jax.ShapeDtypeStruct((_H, _S, _S1), jnp.bfloat16),
    )(cb[: _H * _S1].reshape(_H, _S1, _S1), staf)

    att = pl.pallas_call(
        _attn_body,
        grid=(_H, _S // _BQ),
        in_specs=[pl.BlockSpec((_BQ, _HD), lambda h, i: (i, h)),
                  pl.BlockSpec((_S, _HD), lambda h, i: (0, h)),
                  pl.BlockSpec((_S, _HD), lambda h, i: (0, h)),
                  pl.BlockSpec((1, _S, _S1), lambda h, i: (h, 0, 0))],
        out_specs=pl.BlockSpec((_BQ, _HD), lambda h, i: (i, h)),
        out_shape=jax.ShapeDtypeStruct((_S, _C), jnp.bfloat16),
    )(q2, k2, v2, bias)

    out = pl.pallas_call(
        _proj_body,
        grid=(_S // _BX,),
        in_specs=[tile_x, full_cc, row_c],
        out_specs=tile_x,
        out_shape=jax.ShapeDtypeStruct((_S, _C), f32),
    )(att, Wo.T.astype(jnp.bfloat16), bo.reshape(1, _C))

    return out.reshape(_B, _S, _C)


# speedup vs baseline: 1.3477x; 1.3154x over previous
import math

import jax
import jax.numpy as jnp
import numpy as np
from jax.experimental import pallas as pl
from jax.experimental.pallas import tpu as pltpu
from jax.experimental.pallas import tpu_sc as plsc

_PERM_HD = np.concatenate([np.arange(0, 128, 2), np.arange(1, 128, 2)])
_PERM_C = np.concatenate([h * 128 + _PERM_HD for h in range(6)])

_B, _S, _C, _HD = 1, 4096, 768, 128
_H = _C // _HD
_S1 = _S // 64
_PTHR = 0.9
_NEG = -1e30
_BQ = 1024
_SLAB = 256
_BX = 512


def _qkv_body(x_ref, wq_ref, bq_ref, wk_ref, bk_ref, wv_ref, bv_ref,
              gq_ref, gk_ref, r00_ref, r01_ref, r10_ref, r11_ref,
              q_ref, k_ref, v_ref, qa_ref, ka_ref):
    x = x_ref[...]
    eps = jnp.finfo(jnp.float32).eps

    def norm_rope(t, g_ref):
        t3 = t.reshape(_BX, _H, _HD)
        ms = jnp.mean(t3 * t3, axis=-1, keepdims=True)
        t3 = t3 * jax.lax.rsqrt(ms + eps) * g_ref[...].reshape(1, 1, _HD)
        e = t3[:, :, : _HD // 2]
        o = t3[:, :, _HD // 2:]
        r00 = r00_ref[...][:, None, :]
        r01 = r01_ref[...][:, None, :]
        r10 = r10_ref[...][:, None, :]
        r11 = r11_ref[...][:, None, :]
        oe = r00 * e + r01 * o
        oo = r10 * e + r11 * o
        return jnp.concatenate([oe, oo], axis=-1).reshape(_BX, _C)

    xb = x.astype(jnp.bfloat16)
    q = jnp.dot(xb, wq_ref[...], preferred_element_type=jnp.float32) + bq_ref[...]
    k = jnp.dot(xb, wk_ref[...], preferred_element_type=jnp.float32) + bk_ref[...]
    v = jnp.dot(xb, wv_ref[...], preferred_element_type=jnp.float32) + bv_ref[...]
    qf = norm_rope(q, gq_ref)
    kf = norm_rope(k, gk_ref)
    q_ref[...] = qf.astype(jnp.bfloat16)
    k_ref[...] = kf.astype(jnp.bfloat16)
    v_ref[...] = v.astype(jnp.bfloat16)
    qa_ref[...] = jnp.mean(qf.reshape(_BX // 64, 64, _C), axis=1)
    ka_ref[...] = jnp.mean(kf.reshape(_BX // 64, 64, _C), axis=1)


def _amap_body(qa_ref, ka_ref, a_ref):
    qa = qa_ref[...]
    ka = ka_ref[...]
    logits = jax.lax.dot_general(
        qa, ka, (((1,), (1,)), ((), ())),
        preferred_element_type=jnp.float32) * (1.0 / math.sqrt(_HD))
    a_ref[0] = jax.nn.softmax(logits, axis=-1)


_RPW = 16
_NW = 32
_SCROWS = _RPW * _NW


def _sc_vsort(a):
    k, i = plsc.sort_key_val(a[0], a[1])
    return (k, i)


def _sc_rev(a):
    return (jax.lax.rev(a[0], (0,)), jax.lax.rev(a[1], (0,)))


def _sc_cmpx(a, b):
    m = a[0] <= b[0]
    lo = (jnp.where(m, a[0], b[0]), jnp.where(m, a[1], b[1]))
    hi = (jnp.where(m, b[0], a[0]), jnp.where(m, b[1], a[1]))
    return lo, hi


def _sc_sort64(vs):
    s = [_sc_vsort(v) for v in vs]
    a_lo, a_hi = _sc_cmpx(s[0], _sc_rev(s[1]))
    b_lo, b_hi = _sc_cmpx(s[2], _sc_rev(s[3]))
    a = (_sc_vsort(a_lo), _sc_vsort(a_hi))
    b = (_sc_vsort(b_lo), _sc_vsort(b_hi))
    l0, h0 = _sc_cmpx(a[0], _sc_rev(b[1]))
    l1, h1 = _sc_cmpx(a[1], _sc_rev(b[0]))
    out = []
    for x0, x1 in ((l0, l1), (h0, h1)):
        p, q = _sc_cmpx(x0, x1)
        out.append(_sc_vsort(p))
        out.append(_sc_vsort(q))
    return out


def _sc_mask_body(amap_hbm, out_hbm, amap_v, bias_v):
    wid = jax.lax.axis_index("s") * 2 + jax.lax.axis_index("c")
    base = wid * _RPW
    pltpu.sync_copy(amap_hbm.at[pl.ds(base, _RPW)], amap_v)
    thr = 1.0 - _PTHR
    for r in range(_RPW):
        vs = [(amap_v[r, pl.ds(16 * t, 16)],
               jax.lax.iota(jnp.int32, 16) + 16 * t) for t in range(4)]
        srt = _sc_sort64(vs)
        carry = 0.0
        for t in range(4):
            k, i = srt[t]
            c = plsc.cumsum(k) + carry
            carry = carry + jnp.sum(k)
            vals = jnp.where(c >= thr, 0.0, _NEG)
            plsc.store_scatter(
                bias_v, [jnp.full((16,), r, jnp.int32), i], vals)
    pltpu.sync_copy(bias_v, out_hbm.at[pl.ds(base, _RPW)])


def _expand_body(cb_ref, sta_ref, bias_ref):
    cb = jnp.where(sta_ref[0] > 0.0, 0.0, cb_ref[0])
    bcol = cb.astype(jnp.bfloat16).T
    bias_ref[0] = jnp.broadcast_to(
        bcol[:, None, :], (_S1, 64, _S1)).reshape(_S, _S1)


def _attn_body(q_ref, k_ref, v_ref, bias_ref, o_ref):
    i = pl.program_id(1)
    k = k_ref[...]
    v = v_ref[...]
    k_aug = jnp.concatenate([k, bias_ref[0]], axis=1)
    for s0 in range(0, _BQ, _SLAB):
        q = q_ref[pl.ds(s0, _SLAB), :]
        blk0 = i * (_BQ // 64) + s0 // 64
        r_blk = (jax.lax.broadcasted_iota(jnp.int32, (_SLAB, _S1), 0) // 64
                 + blk0)
        c_idx = jax.lax.broadcasted_iota(jnp.int32, (_SLAB, _S1), 1)
        onehot = (c_idx == r_blk).astype(jnp.bfloat16)
        q_aug = jnp.concatenate([q, onehot], axis=1)
        s = jax.lax.dot_general(
            q_aug, k_aug, (((1,), (1,)), ((), ())),
            preferred_element_type=jnp.float32)
        m = jnp.max(s, axis=-1, keepdims=True)
        p = jnp.exp((s - m) * (1.0 / math.sqrt(_HD)))
        l = jnp.sum(p, axis=-1, keepdims=True)
        o = jnp.dot(p.astype(jnp.bfloat16), v,
                    preferred_element_type=jnp.float32) / l
        o_ref[pl.ds(s0, _SLAB), :] = o.astype(jnp.bfloat16)


def _proj_body(x_ref, w_ref, b_ref, o_ref):
    o_ref[...] = (jnp.dot(x_ref[...], w_ref[...],
                          preferred_element_type=jnp.float32) + b_ref[...])


def kernel(x, rope, sta_mask, Wq, bq, Wk, bk, Wv, bv, gq, gk, Wo, bo):
    f32 = jnp.float32
    x2 = x.reshape(_S, _C)
    rr = rope.reshape(_S, _HD // 2, 2, 2)
    r00, r01 = rr[:, :, 0, 0], rr[:, :, 0, 1]
    r10, r11 = rr[:, :, 1, 0], rr[:, :, 1, 1]
    staf = sta_mask.reshape(_H, _S1, _S1).astype(f32)

    full_cc = pl.BlockSpec((_C, _C), lambda i: (0, 0))
    row_c = pl.BlockSpec((1, _C), lambda i: (0, 0))
    tile_x = pl.BlockSpec((_BX, _C), lambda i: (i, 0))
    tile_r = pl.BlockSpec((_BX, _HD // 2), lambda i: (i, 0))
    g_spec = pl.BlockSpec((1, _HD), lambda i: (0, 0))

    pool_tile = pl.BlockSpec((_BX // 64, _C), lambda i: (i, 0))
    q2, k2, v2, qa2, ka2 = pl.pallas_call(
        _qkv_body,
        grid=(_S // _BX,),
        in_specs=[tile_x, full_cc, row_c, full_cc, row_c, full_cc, row_c,
                  g_spec, g_spec, tile_r, tile_r, tile_r, tile_r],
        out_specs=[tile_x, tile_x, tile_x, pool_tile, pool_tile],
        out_shape=[jax.ShapeDtypeStruct((_S, _C), jnp.bfloat16)] * 3
        + [jax.ShapeDtypeStruct((_S1, _C), f32)] * 2,
    )(x2, Wq.T[:, _PERM_C].astype(jnp.bfloat16), bq[_PERM_C].reshape(1, _C),
      Wk.T[:, _PERM_C].astype(jnp.bfloat16), bk[_PERM_C].reshape(1, _C),
      Wv.T.astype(jnp.bfloat16), bv.reshape(1, _C),
      gq[_PERM_HD].reshape(1, _HD), gk[_PERM_HD].reshape(1, _HD),
      r00, r01, r10, r11)

    head_col = pl.BlockSpec((_S, _HD), lambda h: (0, h))
    pool_col = pl.BlockSpec((_S1, _HD), lambda h: (0, h))
    blk_spec = pl.BlockSpec((1, _S1, _S1), lambda h: (h, 0, 0))
    bias_spec = pl.BlockSpec((1, _S, _S1), lambda h: (h, 0, 0))
    amap = pl.pallas_call(
        _amap_body,
        grid=(_H,),
        in_specs=[pool_col, pool_col],
        out_specs=blk_spec,
        out_shape=jax.ShapeDtypeStruct((_H, _S1, _S1), f32),
    )(qa2, ka2)

    amap_pad = jnp.zeros((_SCROWS, _S1), f32)
    amap_pad = amap_pad.at[: _H * _S1].set(amap.reshape(_H * _S1, _S1))
    cb = pl.kernel(
        _sc_mask_body,
        out_type=jax.ShapeDtypeStruct((_SCROWS, _S1), f32),
        mesh=plsc.VectorSubcoreMesh(core_axis_name="c", subcore_axis_name="s"),
        scratch_types=[pltpu.VMEM((_RPW, _S1), f32),
                       pltpu.VMEM((_RPW, _S1), f32)],
        compiler_params=pltpu.CompilerParams(needs_layout_passes=False),
    )(amap_pad)

    bias = pl.pallas_call(
        _expand_body,
        grid=(_H,),
        in_specs=[blk_spec, blk_spec],
        out_specs=bias_spec,
        out_shape=jax.ShapeDtypeStruct((_H, _S, _S1), jnp.bfloat16),
    )(cb[: _H * _S1].reshape(_H, _S1, _S1), staf)

    att = pl.pallas_call(
        _attn_body,
        grid=(_H, _S // _BQ),
        in_specs=[pl.BlockSpec((_BQ, _HD), lambda h, i: (i, h)),
                  pl.BlockSpec((_S, _HD), lambda h, i: (0, h)),
                  pl.BlockSpec((_S, _HD), lambda h, i: (0, h)),
                  pl.BlockSpec((1, _S, _S1), lambda h, i: (h, 0, 0))],
        out_specs=pl.BlockSpec((_BQ, _HD), lambda h, i: (i, h)),
        out_shape=jax.ShapeDtypeStruct((_S, _C), jnp.bfloat16),
    )(q2, k2, v2, bias)

    out = pl.pallas_call(
        _proj_body,
        grid=(_S // _BX,),
        in_specs=[tile_x, full_cc, row_c],
        out_specs=tile_x,
        out_shape=jax.ShapeDtypeStruct((_S, _C), f32),
    )(att, Wo.T.astype(jnp.bfloat16), bo.reshape(1, _C))

    return out.reshape(_B, _S, _C)
